# Initial kernel scaffold; baseline (speedup 1.0000x reference)
import jax
import jax.numpy as jnp
from jax.experimental import pallas as pl


def kernel(replayed_states, replayed_rewards, W, b):
    raise NotImplementedError("write your pallas kernel here")


# speedup vs baseline: 24.2555x; 24.2555x over previous
import jax
import jax.numpy as jnp
from jax.experimental import pallas as pl
from jax.experimental.pallas import tpu as pltpu

_STATE_DIM = 128
_SEM_DIM = 64
_SLOTS = 8192
_LR = 0.01
_N = 4096
_BLK = 1024
_INF = float('inf')


def _consolidate_kernel(states_ref, rewards_ref, w_ref, b_ref,
                        traces_ref, num_ref, ms_ref,
                        sem_ref, strengths_ref):
    sem_ref[...] = jax.lax.dot_general(
        states_ref[...], w_ref[...],
        dimension_numbers=(((1,), (1,)), ((), ())),
        preferred_element_type=jnp.float32) + b_ref[...]

    traces_ref[...] = jnp.zeros((_SLOTS, _SEM_DIM), jnp.float32)
    strengths_ref[...] = jnp.zeros((_SLOTS, 1), jnp.float32)

    row_ids = jax.lax.broadcasted_iota(jnp.int32, (_SLOTS, 1), 0)
    blk_ids = jax.lax.broadcasted_iota(jnp.int32, (_BLK, 1), 0)

    def step(i, carry):
        num = carry
        content = sem_ref[pl.ds(i, 1), :]

        def dist_block(k, dcarry):
            dmin, jmin = dcarry
            base = k * _BLK
            rows = traces_ref[pl.ds(base, _BLK), :]
            diffs = rows - content
            d2 = jnp.sum(diffs * diffs, axis=1, keepdims=True)
            ids = base + blk_ids
            d2m = jnp.where(ids < num, d2, _INF)
            bmin = jnp.min(d2m)
            bj = jnp.min(jnp.where(d2m == bmin, ids, _SLOTS))
            take = bmin < dmin
            return (jnp.where(take, bmin, dmin),
                    jnp.where(take, bj, jmin))

        nblk = (num + (_BLK - 1)) // _BLK
        dmin, j = jax.lax.fori_loop(
            0, nblk, dist_block, (jnp.float32(_INF), jnp.int32(0)))
        do_update = (num > 0) & (dmin < 4.0)

        reward = jnp.abs(rewards_ref[pl.ds(i, 1), :][0, 0])
        eff_lr = _LR * (1.0 + reward)
        old = traces_ref[pl.ds(j, 1), :]
        upd = old + (content - old) * eff_lr
        s_old = strengths_ref[pl.ds(j, 1), :]

        tgt = jnp.where(do_update, j, num)
        traces_ref[pl.ds(tgt, 1), :] = jnp.where(do_update, upd, content)
        strengths_ref[pl.ds(tgt, 1), :] = jnp.where(do_update, s_old + 1.0, 1.0)
        return jnp.where(do_update, num, num + 1)

    num = jax.lax.fori_loop(0, _N, step, jnp.int32(0))

    valid = row_ids < num
    total = jnp.sum(jnp.where(valid, strengths_ref[...], 0.0))
    ms = jnp.where(num > 0, total / jnp.maximum(num, 1).astype(jnp.float32), 0.0)
    num_ref[...] = jnp.full((1, 1), num, jnp.int32)
    ms_ref[...] = jnp.full((1, 1), ms, jnp.float32)


@jax.jit
def kernel(replayed_states, replayed_rewards, W, b):
    rewards2 = replayed_rewards.reshape(_N, 1)
    b2 = b.reshape(1, _SEM_DIM)
    traces, num, ms = pl.pallas_call(
        _consolidate_kernel,
        out_shape=[
            jax.ShapeDtypeStruct((_SLOTS, _SEM_DIM), jnp.float32),
            jax.ShapeDtypeStruct((1, 1), jnp.int32),
            jax.ShapeDtypeStruct((1, 1), jnp.float32),
        ],
        scratch_shapes=[
            pltpu.VMEM((_N, _SEM_DIM), jnp.float32),
            pltpu.VMEM((_SLOTS, 1), jnp.float32),
        ],
    )(replayed_states, rewards2, W, b2)
    return (jnp.array(True), jnp.array(_N, jnp.int32), num[0, 0], ms[0, 0],
            traces)
